# baseline (speedup 1.0000x reference)
import jax
import jax.numpy as jnp
from jax import lax
from jax.experimental import pallas as pl
from jax.experimental.pallas import tpu as pltpu
from jax.experimental.pallas import tpu_sc as plsc

B = 256
SEQ = 201
D = 768
M = 100000

_NC = 2
_NS = 16
_ROWS_PER_TILE = B // (_NC * _NS)


def _mesh():
  return plsc.VectorSubcoreMesh(core_axis_name="c", subcore_axis_name="s",
                                num_cores=_NC, num_subcores=_NS)


def _sc_gather_body(table_hbm, idx_hbm, out_hbm, idx_v, rows_v, sem):
  wid = lax.axis_index("s") * _NC + lax.axis_index("c")
  base = wid * _ROWS_PER_TILE
  pltpu.sync_copy(idx_hbm.at[pl.ds(base, _ROWS_PER_TILE)], idx_v)
  pltpu.async_copy(table_hbm.at[idx_v], rows_v, sem).wait()
  pltpu.sync_copy(rows_v, out_hbm.at[pl.ds(base, _ROWS_PER_TILE)])


def _sc_gather(table, idx):
  gather = pl.kernel(
      _sc_gather_body,
      out_type=jax.ShapeDtypeStruct((B, D), jnp.float32),
      mesh=_mesh(),
      scratch_types=[
          pltpu.VMEM((_ROWS_PER_TILE,), jnp.int32),
          pltpu.VMEM((_ROWS_PER_TILE, D), jnp.float32),
          pltpu.SemaphoreType.DMA,
      ],
  )
  return gather(table, idx)


def _sc_scatter(afu, gsend, idx):
  def stage(refs):
    afu_ref, gsend_ref, idx_ref = refs

    @pl.core_map(_mesh())
    def _():
      def inner(idx_v, rows_v, sem):
        wid = lax.axis_index("s") * _NC + lax.axis_index("c")
        base = wid * _ROWS_PER_TILE
        pltpu.sync_copy(idx_ref.at[pl.ds(base, _ROWS_PER_TILE)], idx_v)
        pltpu.sync_copy(gsend_ref.at[pl.ds(base, _ROWS_PER_TILE)], rows_v)
        pltpu.async_copy(rows_v, afu_ref.at[idx_v], sem).wait()

      pl.run_scoped(inner,
                    pltpu.VMEM((_ROWS_PER_TILE,), jnp.int32),
                    pltpu.VMEM((_ROWS_PER_TILE, D), jnp.float32),
                    pltpu.SemaphoreType.DMA)

  afu_out, _, _ = pl.run_state(stage)((afu, gsend, idx))
  return afu_out


def _compute_body(t_ref, g_ref, wt_ref, bt_ref, wg_ref, bg_ref,
                  ic_ref, ir_ref, text_any, trow_out, gsend_out):
  del text_any
  t = t_ref[...]
  g = g_ref[...]
  tt = lax.dot_general(t, wt_ref[...], (((1,), (1,)), ((), ())),
                       preferred_element_type=jnp.float32) + bt_ref[...]
  a = jnp.sum(t * tt, axis=1, keepdims=True)
  b = jnp.sum(g * t, axis=1, keepdims=True)
  m = jnp.maximum(a, b)
  ea = jnp.exp(a - m)
  eb = jnp.exp(b - m)
  s = ea + eb
  trow_out[...] = (ea / s) * t + (eb / s) * g

  gt = lax.dot_general(g, wg_ref[...], (((1,), (1,)), ((), ())),
                       preferred_element_type=jnp.float32) + bg_ref[...]
  c = jnp.sum(gt * g, axis=1, keepdims=True)
  m2 = jnp.maximum(c, b)
  ec = jnp.exp(c - m2)
  ed = jnp.exp(b - m2)
  s2 = ec + ed
  graph = (ec / s2) * g + (ed / s2) * t

  eqf = (ic_ref[...] == ir_ref[...]).astype(jnp.float32)
  ki = lax.broadcasted_iota(jnp.int32, (B, B), 0)
  ji = lax.broadcasted_iota(jnp.int32, (B, B), 1)
  upper = (ki > ji).astype(jnp.float32)
  suffix = lax.dot_general(eqf, upper, (((1,), (0,)), ((), ())),
                           preferred_element_type=jnp.float32)
  sel = eqf * (suffix == 0).astype(jnp.float32)
  gsend_out[...] = lax.dot_general(sel, graph, (((1,), (0,)), ((), ())),
                                   preferred_element_type=jnp.float32)


_compute = pl.pallas_call(
    _compute_body,
    grid=(1,),
    in_specs=[
        pl.BlockSpec((B, D), lambda i: (0, 0)),
        pl.BlockSpec((B, D), lambda i: (0, 0)),
        pl.BlockSpec((D, D), lambda i: (0, 0)),
        pl.BlockSpec((1, D), lambda i: (0, 0)),
        pl.BlockSpec((D, D), lambda i: (0, 0)),
        pl.BlockSpec((1, D), lambda i: (0, 0)),
        pl.BlockSpec((B, 1), lambda i: (0, 0)),
        pl.BlockSpec((1, B), lambda i: (0, 0)),
        pl.BlockSpec(memory_space=pl.ANY),
    ],
    out_specs=(
        pl.BlockSpec((B, D), lambda i: (0, 0)),
        pl.BlockSpec((B, D), lambda i: (0, 0)),
    ),
    out_shape=(
        jax.ShapeDtypeStruct((B, SEQ * D), jnp.float32),
        jax.ShapeDtypeStruct((B, D), jnp.float32),
    ),
    input_output_aliases={8: 0},
)


def kernel(text, all_user_feature, user_neighbor_index, W_text, b_text,
           W_graph, b_graph):
  idx = user_neighbor_index[:, 0].astype(jnp.int32)
  text_ini = text[:, 0, :]

  graph_ini = _sc_gather(all_user_feature, idx)

  text2d, gsend = _compute(
      text_ini, graph_ini, W_text, b_text.reshape(1, D), W_graph,
      b_graph.reshape(1, D), idx.reshape(B, 1), idx.reshape(1, B),
      text.reshape(B, SEQ * D))

  afu_out = _sc_scatter(all_user_feature, gsend, idx)
  return (text2d.reshape(B, SEQ, D), afu_out)


# speedup vs baseline: 1.4842x; 1.4842x over previous
import jax
import jax.numpy as jnp
from jax import lax
from jax.experimental import pallas as pl
from jax.experimental.pallas import tpu as pltpu
from jax.experimental.pallas import tpu_sc as plsc

B = 256
SEQ = 201
D = 768
M = 100000

_NC = 2
_NS = 16
_ROWS_PER_TILE = B // (_NC * _NS)


def _mesh():
  return plsc.VectorSubcoreMesh(core_axis_name="c", subcore_axis_name="s",
                                num_cores=_NC, num_subcores=_NS)


def _sc_gather_body(table_hbm, idx_hbm, out_hbm, idx_v, rows_v, sem):
  wid = lax.axis_index("s") * _NC + lax.axis_index("c")
  base = wid * _ROWS_PER_TILE
  pltpu.sync_copy(idx_hbm.at[pl.ds(base, _ROWS_PER_TILE)], idx_v)
  pltpu.async_copy(table_hbm.at[idx_v], rows_v, sem).wait()
  pltpu.sync_copy(rows_v, out_hbm.at[pl.ds(base, _ROWS_PER_TILE)])


def _sc_gather(table, idx):
  gather = pl.kernel(
      _sc_gather_body,
      out_type=jax.ShapeDtypeStruct((B, D), jnp.float32),
      mesh=_mesh(),
      scratch_types=[
          pltpu.VMEM((_ROWS_PER_TILE,), jnp.int32),
          pltpu.VMEM((_ROWS_PER_TILE, D), jnp.float32),
          pltpu.SemaphoreType.DMA,
      ],
  )
  return gather(table, idx)


def _sc_scatter(afu, gsend, idx):
  def stage(refs):
    afu_ref, gsend_ref, idx_ref = refs

    @pl.core_map(_mesh())
    def _():
      def inner(idx_v, rows_v, sem):
        wid = lax.axis_index("s") * _NC + lax.axis_index("c")
        base = wid * _ROWS_PER_TILE
        pltpu.sync_copy(idx_ref.at[pl.ds(base, _ROWS_PER_TILE)], idx_v)
        pltpu.sync_copy(gsend_ref.at[pl.ds(base, _ROWS_PER_TILE)], rows_v)
        pltpu.async_copy(rows_v, afu_ref.at[idx_v], sem).wait()

      pl.run_scoped(inner,
                    pltpu.VMEM((_ROWS_PER_TILE,), jnp.int32),
                    pltpu.VMEM((_ROWS_PER_TILE, D), jnp.float32),
                    pltpu.SemaphoreType.DMA)

  afu_out, _, _ = pl.run_state(stage)((afu, gsend, idx))
  return afu_out


def _compute_body(g_ref, wt_ref, bt_ref, wg_ref, bg_ref,
                  ic_ref, ir_ref, text_any, text_out, gsend_out,
                  tini_v, tnew_v, semi, semo):
  del text_any
  pltpu.make_async_copy(text_out.at[:, 0, :], tini_v, semi).start()

  g = g_ref[...]
  gt = lax.dot_general(g, wg_ref[...], (((1,), (1,)), ((), ())),
                       preferred_element_type=jnp.float32) + bg_ref[...]
  c = jnp.sum(gt * g, axis=1, keepdims=True)

  eqf = (ic_ref[...] == ir_ref[...]).astype(jnp.float32)
  ki = lax.broadcasted_iota(jnp.int32, (B, B), 0)
  ji = lax.broadcasted_iota(jnp.int32, (B, B), 1)
  upper = (ki > ji).astype(jnp.float32)
  suffix = lax.dot_general(eqf, upper, (((1,), (0,)), ((), ())),
                           preferred_element_type=jnp.float32)
  sel = eqf * (suffix == 0).astype(jnp.float32)

  pltpu.make_async_copy(text_out.at[:, 0, :], tini_v, semi).wait()
  t = tini_v[...]
  tt = lax.dot_general(t, wt_ref[...], (((1,), (1,)), ((), ())),
                       preferred_element_type=jnp.float32) + bt_ref[...]
  a = jnp.sum(t * tt, axis=1, keepdims=True)
  b = jnp.sum(g * t, axis=1, keepdims=True)
  m = jnp.maximum(a, b)
  ea = jnp.exp(a - m)
  eb = jnp.exp(b - m)
  s = ea + eb
  tnew_v[...] = (ea / s) * t + (eb / s) * g
  out_dma = pltpu.make_async_copy(tnew_v, text_out.at[:, 0, :], semo)
  out_dma.start()

  m2 = jnp.maximum(c, b)
  ec = jnp.exp(c - m2)
  ed = jnp.exp(b - m2)
  s2 = ec + ed
  graph = (ec / s2) * g + (ed / s2) * t
  gsend_out[...] = lax.dot_general(sel, graph, (((1,), (0,)), ((), ())),
                                   preferred_element_type=jnp.float32)
  out_dma.wait()


_compute = pl.pallas_call(
    _compute_body,
    grid=(1,),
    in_specs=[
        pl.BlockSpec((B, D), lambda i: (0, 0)),
        pl.BlockSpec((D, D), lambda i: (0, 0)),
        pl.BlockSpec((1, D), lambda i: (0, 0)),
        pl.BlockSpec((D, D), lambda i: (0, 0)),
        pl.BlockSpec((1, D), lambda i: (0, 0)),
        pl.BlockSpec((B, 1), lambda i: (0, 0)),
        pl.BlockSpec((1, B), lambda i: (0, 0)),
        pl.BlockSpec(memory_space=pl.ANY),
    ],
    out_specs=(
        pl.BlockSpec(memory_space=pl.ANY),
        pl.BlockSpec((B, D), lambda i: (0, 0)),
    ),
    out_shape=(
        jax.ShapeDtypeStruct((B, SEQ, D), jnp.float32),
        jax.ShapeDtypeStruct((B, D), jnp.float32),
    ),
    scratch_shapes=[
        pltpu.VMEM((B, D), jnp.float32),
        pltpu.VMEM((B, D), jnp.float32),
        pltpu.SemaphoreType.DMA,
        pltpu.SemaphoreType.DMA,
    ],
    input_output_aliases={7: 0},
)


def kernel(text, all_user_feature, user_neighbor_index, W_text, b_text,
           W_graph, b_graph):
  idx = user_neighbor_index[:, 0].astype(jnp.int32)

  graph_ini = _sc_gather(all_user_feature, idx)

  text_out, gsend = _compute(
      graph_ini, W_text, b_text.reshape(1, D), W_graph,
      b_graph.reshape(1, D), idx.reshape(B, 1), idx.reshape(1, B), text)

  afu_out = _sc_scatter(all_user_feature, gsend, idx)
  return (text_out, afu_out)


# speedup vs baseline: 2.2392x; 1.5087x over previous
import jax
import jax.numpy as jnp
from jax import lax
from jax.experimental import pallas as pl
from jax.experimental.pallas import tpu as pltpu
from jax.experimental.pallas import tpu_sc as plsc

B = 256
SEQ = 201
D = 768
M = 100000

_NC = 2
_NS = 16
_ROWS_PER_TILE = B // (_NC * _NS)


def _mesh():
  return plsc.VectorSubcoreMesh(core_axis_name="c", subcore_axis_name="s",
                                num_cores=_NC, num_subcores=_NS)


def _sc_gather_body(table_hbm, idx_hbm, out_hbm, idx_v, rows_v, sem):
  wid = lax.axis_index("s") * _NC + lax.axis_index("c")
  base = wid * _ROWS_PER_TILE
  pltpu.sync_copy(idx_hbm.at[pl.ds(base, _ROWS_PER_TILE)], idx_v)
  pltpu.async_copy(table_hbm.at[idx_v], rows_v, sem).wait()
  pltpu.sync_copy(rows_v, out_hbm.at[pl.ds(base, _ROWS_PER_TILE)])


def _sc_gather(table, idx):
  gather = pl.kernel(
      _sc_gather_body,
      out_type=jax.ShapeDtypeStruct((B, D), jnp.float32),
      mesh=_mesh(),
      scratch_types=[
          pltpu.VMEM((_ROWS_PER_TILE,), jnp.int32),
          pltpu.VMEM((_ROWS_PER_TILE, D), jnp.float32),
          pltpu.SemaphoreType.DMA,
      ],
  )
  return gather(table, idx)


def _sc_scatter(afu, gsend, idx):
  def stage(refs):
    afu_ref, gsend_ref, idx_ref = refs

    @pl.core_map(_mesh())
    def _():
      def inner(idx_v, rows_v, sem):
        wid = lax.axis_index("s") * _NC + lax.axis_index("c")
        base = wid * _ROWS_PER_TILE
        pltpu.sync_copy(idx_ref.at[pl.ds(base, _ROWS_PER_TILE)], idx_v)
        pltpu.sync_copy(gsend_ref.at[pl.ds(base, _ROWS_PER_TILE)], rows_v)
        pltpu.async_copy(rows_v, afu_ref.at[idx_v], sem).wait()

      pl.run_scoped(inner,
                    pltpu.VMEM((_ROWS_PER_TILE,), jnp.int32),
                    pltpu.VMEM((_ROWS_PER_TILE, D), jnp.float32),
                    pltpu.SemaphoreType.DMA)

  afu_out, _, _ = pl.run_state(stage)((afu, gsend, idx))
  return afu_out


def _compute_body(g_ref, wt_ref, bt_ref, wg_ref, bg_ref,
                  ic_ref, ir_ref, text_any, text_out, gsend_out,
                  tini_v, tnew_v, semi, semo):
  del text_any
  pltpu.make_async_copy(text_out.at[0], tini_v, semi).start()

  g = g_ref[...]
  gt = lax.dot_general(g, wg_ref[...], (((1,), (1,)), ((), ())),
                       preferred_element_type=jnp.float32) + bg_ref[...]
  c = jnp.sum(gt * g, axis=1, keepdims=True)

  eqf = (ic_ref[...] == ir_ref[...]).astype(jnp.float32)
  ki = lax.broadcasted_iota(jnp.int32, (B, B), 0)
  ji = lax.broadcasted_iota(jnp.int32, (B, B), 1)
  upper = (ki > ji).astype(jnp.float32)
  suffix = lax.dot_general(eqf, upper, (((1,), (0,)), ((), ())),
                           preferred_element_type=jnp.float32)
  sel = eqf * (suffix == 0).astype(jnp.float32)

  pltpu.make_async_copy(text_out.at[0], tini_v, semi).wait()
  t = tini_v[...]
  tt = lax.dot_general(t, wt_ref[...], (((1,), (1,)), ((), ())),
                       preferred_element_type=jnp.float32) + bt_ref[...]
  a = jnp.sum(t * tt, axis=1, keepdims=True)
  b = jnp.sum(g * t, axis=1, keepdims=True)
  m = jnp.maximum(a, b)
  ea = jnp.exp(a - m)
  eb = jnp.exp(b - m)
  s = ea + eb
  tnew_v[...] = (ea / s) * t + (eb / s) * g
  out_dma = pltpu.make_async_copy(tnew_v, text_out.at[0], semo)
  out_dma.start()

  m2 = jnp.maximum(c, b)
  ec = jnp.exp(c - m2)
  ed = jnp.exp(b - m2)
  s2 = ec + ed
  graph = (ec / s2) * g + (ed / s2) * t
  gsend_out[...] = lax.dot_general(sel, graph, (((1,), (0,)), ((), ())),
                                   preferred_element_type=jnp.float32)
  out_dma.wait()


_compute = pl.pallas_call(
    _compute_body,
    grid=(1,),
    in_specs=[
        pl.BlockSpec((B, D), lambda i: (0, 0)),
        pl.BlockSpec((D, D), lambda i: (0, 0)),
        pl.BlockSpec((1, D), lambda i: (0, 0)),
        pl.BlockSpec((D, D), lambda i: (0, 0)),
        pl.BlockSpec((1, D), lambda i: (0, 0)),
        pl.BlockSpec((B, 1), lambda i: (0, 0)),
        pl.BlockSpec((1, B), lambda i: (0, 0)),
        pl.BlockSpec(memory_space=pl.ANY),
    ],
    out_specs=(
        pl.BlockSpec(memory_space=pl.ANY),
        pl.BlockSpec((B, D), lambda i: (0, 0)),
    ),
    out_shape=(
        jax.ShapeDtypeStruct((SEQ, B, D), jnp.float32),
        jax.ShapeDtypeStruct((B, D), jnp.float32),
    ),
    scratch_shapes=[
        pltpu.VMEM((B, D), jnp.float32),
        pltpu.VMEM((B, D), jnp.float32),
        pltpu.SemaphoreType.DMA,
        pltpu.SemaphoreType.DMA,
    ],
    input_output_aliases={7: 0},
)


def kernel(text, all_user_feature, user_neighbor_index, W_text, b_text,
           W_graph, b_graph):
  idx = user_neighbor_index[:, 0].astype(jnp.int32)

  graph_ini = _sc_gather(all_user_feature, idx)

  text_t = jnp.transpose(text, (1, 0, 2))
  text_out_t, gsend = _compute(
      graph_ini, W_text, b_text.reshape(1, D), W_graph,
      b_graph.reshape(1, D), idx.reshape(B, 1), idx.reshape(1, B), text_t)

  afu_out = _sc_scatter(all_user_feature, gsend, idx)
  return (jnp.transpose(text_out_t, (1, 0, 2)), afu_out)
